# baseline (speedup 1.0000x reference)
import jax
import jax.numpy as jnp
from jax import lax
from jax.experimental import pallas as pl
from jax.experimental.pallas import tpu as pltpu
from jax.experimental.pallas import tpu_sc as plsc

_N = 10000
_D = 128
_H = 64
_E = 320000
_CLS = 10
_EPS = 1e-12

_C = 128
_NCH = _E // _C
_KB = 2
_NCHK = _NCH // _KB
_NS = 16
_T = -(-_NCHK // _NS)
_RPT = _N // _NS
_ZR = 125

_BN = 1000
_NB = _N // _BN


_DN = (((1,), (1,)), ((), ()))


def _dot_t(a, b):
    return lax.dot_general(a, b, _DN, preferred_element_type=jnp.float32)


def _tc_pre_body(x_ref, wmi_ref, wmo_ref, ws_ref,
                 hmi0_ref, hmi1_ref, hmo0_ref, hmo1_ref, sh_ref):
    xb = x_ref[...]
    hmi0_ref[...] = _dot_t(xb, wmi_ref[0:_H, :])
    hmi1_ref[...] = _dot_t(xb, wmi_ref[_H:_D, :])
    hmo0_ref[...] = _dot_t(xb, wmo_ref[0:_H, :])
    hmo1_ref[...] = _dot_t(xb, wmo_ref[_H:_D, :])
    sh_ref[...] = _dot_t(xb, ws_ref[...])


def _tc_pre(x, wmi, wmo, ws):
    f32 = jnp.float32
    row = pl.BlockSpec((_BN, _D), lambda i: (i, 0))
    half = pl.BlockSpec((_BN, _H), lambda i: (i, 0))
    full_w = pl.BlockSpec((_D, _D), lambda i: (0, 0))
    return pl.pallas_call(
        _tc_pre_body,
        grid=(_NB,),
        in_specs=[row, full_w, full_w, full_w],
        out_specs=[half, half, half, half, row],
        out_shape=[jax.ShapeDtypeStruct((_N, _H), f32)] * 4
        + [jax.ShapeDtypeStruct((_N, _D), f32)],
    )(x, wmi, wmo, ws)


def _combine(pin0, pin1, pout0, pout1, sh_ref, bmi, bmo, bsi, bso, cin, cout):
    sh = sh_ref[...]
    pin = jnp.concatenate([pin0[...], pin1[...]], axis=1)
    pout = jnp.concatenate([pout0[...], pout1[...]], axis=1)
    ic = pin + bmi[...] + sh + bsi[...]
    oc = pout + bmo[...] + sh + bso[...]
    return jnp.tanh(cin[...] * ic + cout[...] * oc)


def _tc_mid_body(pin0, pin1, pout0, pout1, sh_ref, bmi, bmo, bsi, bso,
                 cin, cout, wmi_ref, wmo_ref, ws_ref,
                 hmi0_ref, hmi1_ref, hmo0_ref, hmo1_ref, sh_o):
    h = _combine(pin0, pin1, pout0, pout1, sh_ref, bmi, bmo, bsi, bso, cin, cout)
    hmi0_ref[...] = _dot_t(h, wmi_ref[0:_H, :])
    hmi1_ref[...] = _dot_t(h, wmi_ref[_H:_D, :])
    hmo0_ref[...] = _dot_t(h, wmo_ref[0:_H, :])
    hmo1_ref[...] = _dot_t(h, wmo_ref[_H:_D, :])
    sh_o[...] = _dot_t(h, ws_ref[...])


def _tc_mid(pin0, pin1, pout0, pout1, sh, bmi, bmo, bsi, bso, cin, cout,
            wmi, wmo, ws):
    f32 = jnp.float32
    row = pl.BlockSpec((_BN, _D), lambda i: (i, 0))
    half = pl.BlockSpec((_BN, _H), lambda i: (i, 0))
    bias = pl.BlockSpec((1, _D), lambda i: (0, 0))
    cvec = pl.BlockSpec((_BN, 1), lambda i: (i, 0))
    full_w = pl.BlockSpec((_D, _D), lambda i: (0, 0))
    return pl.pallas_call(
        _tc_mid_body,
        grid=(_NB,),
        in_specs=[half, half, half, half, row, bias, bias, bias, bias,
                  cvec, cvec, full_w, full_w, full_w],
        out_specs=[half, half, half, half, row],
        out_shape=[jax.ShapeDtypeStruct((_N, _H), f32)] * 4
        + [jax.ShapeDtypeStruct((_N, _D), f32)],
    )(pin0, pin1, pout0, pout1, sh, bmi, bmo, bsi, bso, cin, cout, wmi, wmo, ws)


def _tc_post_body(pin0, pin1, pout0, pout1, sh_ref, bmi, bmo, bsi, bso,
                  cin, cout, wdec_ref, bdec_ref, logp_ref, emb_ref):
    h = _combine(pin0, pin1, pout0, pout1, sh_ref, bmi, bmo, bsi, bso, cin, cout)
    logits = _dot_t(h, wdec_ref[...]) + bdec_ref[...]
    m = jnp.max(logits, axis=-1, keepdims=True)
    e = jnp.exp(logits - m)
    lse = jnp.log(jnp.sum(e, axis=-1, keepdims=True)) + m
    logp_ref[...] = logits - lse
    nrm = jnp.sqrt(jnp.sum(h * h, axis=-1, keepdims=True))
    emb_ref[...] = h / (nrm + _EPS)


def _tc_post(pin0, pin1, pout0, pout1, sh, bmi, bmo, bsi, bso, cin, cout,
             wdec, bdec):
    f32 = jnp.float32
    row = pl.BlockSpec((_BN, _D), lambda i: (i, 0))
    half = pl.BlockSpec((_BN, _H), lambda i: (i, 0))
    bias = pl.BlockSpec((1, _D), lambda i: (0, 0))
    cvec = pl.BlockSpec((_BN, 1), lambda i: (i, 0))
    return pl.pallas_call(
        _tc_post_body,
        grid=(_NB,),
        in_specs=[half, half, half, half, row, bias, bias, bias, bias,
                  cvec, cvec,
                  pl.BlockSpec((_CLS, _D), lambda i: (0, 0)),
                  pl.BlockSpec((1, _CLS), lambda i: (0, 0))],
        out_specs=[pl.BlockSpec((_BN, _CLS), lambda i: (i, 0)), row],
        out_shape=[jax.ShapeDtypeStruct((_N, _CLS), f32),
                   jax.ShapeDtypeStruct((_N, _D), f32)],
    )(pin0, pin1, pout0, pout1, sh, bmi, bmo, bsi, bso, cin, cout, wdec, bdec)


def _sc_body(hmi0, hmi1, hmo0, hmo1, si2, di2, wi, so2, do2, wo,
             pin0, pin1, pout0, pout1,
             acc, tab_s, eib0, eib1, dstb0, dstb1, wvb0, wvb1,
             rows0, rows1,
             sei0, sei1, sdt0, sdt1, swm0, swm1, sg0, sg1, ss0, ss1):
    c = lax.axis_index("c")
    s = lax.axis_index("s")
    ebs = (eib0, eib1)
    dbs = (dstb0, dstb1)
    wbs = (wvb0, wvb1)
    rbs = (rows0, rows1)
    sei = (sei0, sei1)
    sdt = (sdt0, sdt1)
    swm = (swm0, swm1)
    sg = (sg0, sg1)
    ss = (ss0, ss1)

    zeros16 = jnp.zeros((16,), jnp.float32)
    base_r = s * _RPT
    _RB = _KB * _C

    def zero_acc():
        def zinit(i, carry):
            for k4 in range(_H // 16):
                rows0[i, k4 * 16:(k4 + 1) * 16] = zeros16
            return carry

        lax.fori_loop(0, _RB, zinit, 0)
        pltpu.sync_copy(rows0, acc.at[pl.ds(base_r, _RB)])
        pltpu.sync_copy(rows0, acc.at[pl.ds(base_r + _RB, _RB)])
        pltpu.sync_copy(rows0.at[pl.ds(0, _RPT - 2 * _RB)],
                        acc.at[pl.ds(base_r + 2 * _RB, _RPT - 2 * _RB)])

    def run_edges(src_h, dst_h, w_h, tab_h):
        pltpu.sync_copy(tab_h.at[pl.ds(base_r, _RPT)],
                        tab_s.at[pl.ds(base_r, _RPT)])
        plsc.subcore_barrier()
        def cbase(j):
            return jnp.minimum(j, _NCHK - 1) * _KB

        def src_issue(j, b):
            pltpu.async_copy(src_h.at[pl.ds(cbase(j), _KB)], ebs[b], sei[b])

        def src_wait(b):
            pltpu.make_async_copy(src_h.at[pl.ds(0, _KB)], ebs[b], sei[b]).wait()

        def dst_issue(j, b):
            pltpu.async_copy(dst_h.at[pl.ds(cbase(j), _KB)], dbs[b], sdt[b])

        def dst_wait(b):
            pltpu.make_async_copy(dst_h.at[pl.ds(0, _KB)], dbs[b], sdt[b]).wait()

        def w_issue(j, b):
            pltpu.async_copy(w_h.at[pl.ds(cbase(j) * _C, _KB * _C)],
                             wbs[b], swm[b])

        def w_wait(b):
            pltpu.make_async_copy(w_h.at[pl.ds(0, _KB * _C)], wbs[b],
                                  swm[b]).wait()

        def gather_issue(b):
            for k in range(_KB):
                pltpu.async_copy(tab_s.at[ebs[b].at[k]],
                                 rbs[b].at[pl.ds(k * _C, _C)], sg[b])

        def gather_wait(b):
            for k in range(_KB):
                pltpu.make_async_copy(tab_s.at[ebs[b].at[k]],
                                      rbs[b].at[pl.ds(k * _C, _C)],
                                      sg[b]).wait()

        def scatter_issue(b):
            for k in range(_KB):
                pltpu.async_copy(rbs[b].at[pl.ds(k * _C, _C)],
                                 acc.at[dbs[b].at[k]], ss[b], add=True)

        def scatter_wait(b):
            for k in range(_KB):
                pltpu.make_async_copy(rbs[b].at[pl.ds(k * _C, _C)],
                                      acc.at[dbs[b].at[k]], ss[b]).wait()

        src_issue(s, 0)
        src_issue(_NS + s, 1)
        dst_issue(s, 0)
        w_issue(s, 0)
        src_wait(0)
        gather_issue(0)

        def pair(i, carry):
            for b in (0, 1):
                o = b ^ 1
                jj = 2 * i + b
                j = jj * _NS + s
                gather_wait(b)
                src_issue(j + 2 * _NS, b)
                w_issue(j + _NS, o)
                w_wait(b)
                validf = jnp.where(j < _NCHK, 1.0, 0.0).astype(jnp.float32)

                def scale(g, cc):
                    w16 = wbs[b][pl.ds(g * 16, 16)] * validf
                    for e in range(16):
                        wsc = w16[e]
                        r = g * 16 + e
                        for k4 in range(_H // 16):
                            sl = pl.ds(k4 * 16, 16)
                            rbs[b][r, sl] = rbs[b][r, sl] * wsc
                    return cc

                lax.fori_loop(0, _KB * _C // 16, scale, 0, unroll=4)
                if b == 0:
                    @pl.when(i > 0)
                    def _():
                        scatter_wait(1)
                else:
                    scatter_wait(0)
                dst_issue(j + _NS, o)
                src_wait(o)
                gather_issue(o)
                dst_wait(b)
                scatter_issue(b)
            return carry

        lax.fori_loop(0, (_T + 1) // 2, pair, 0)
        scatter_wait(1)
        gather_wait(0)
        src_wait(1)
        dst_wait(0)
        w_wait(0)
        plsc.subcore_barrier()

    zero_acc()

    @pl.when(c == 0)
    def _():
        run_edges(si2, di2, wi, hmi0)
        pltpu.sync_copy(acc.at[pl.ds(base_r, _RPT)], pin0.at[pl.ds(base_r, _RPT)])
        zero_acc()
        run_edges(so2, do2, wo, hmo0)
        pltpu.sync_copy(acc.at[pl.ds(base_r, _RPT)], pout0.at[pl.ds(base_r, _RPT)])

    @pl.when(c == 1)
    def _():
        run_edges(si2, di2, wi, hmi1)
        pltpu.sync_copy(acc.at[pl.ds(base_r, _RPT)], pin1.at[pl.ds(base_r, _RPT)])
        zero_acc()
        run_edges(so2, do2, wo, hmo1)
        pltpu.sync_copy(acc.at[pl.ds(base_r, _RPT)], pout1.at[pl.ds(base_r, _RPT)])


def _sc_propagate(hmi0, hmi1, hmo0, hmo1, si2, di2, wi, so2, do2, wo):
    f32 = jnp.float32
    i32 = jnp.int32
    mesh = plsc.VectorSubcoreMesh(core_axis_name="c", subcore_axis_name="s")
    kfn = pl.kernel(
        _sc_body,
        out_type=[jax.ShapeDtypeStruct((_N, _H), f32)] * 4,
        mesh=mesh,
        compiler_params=pltpu.CompilerParams(use_tc_tiling_on_sc=False),
        scratch_types=[
            pltpu.VMEM_SHARED((_N, _H), f32),
            pltpu.VMEM_SHARED((_N, _H), f32),
            pltpu.VMEM((_KB, _C), i32),
            pltpu.VMEM((_KB, _C), i32),
            pltpu.VMEM((_KB, _C), i32),
            pltpu.VMEM((_KB, _C), i32),
            pltpu.VMEM((_KB * _C,), f32),
            pltpu.VMEM((_KB * _C,), f32),
            pltpu.VMEM((_KB * _C, _H), f32),
            pltpu.VMEM((_KB * _C, _H), f32),
        ] + [pltpu.SemaphoreType.DMA] * 10,
    )
    return kfn(hmi0, hmi1, hmo0, hmo1, si2, di2, wi, so2, do2, wo)


def kernel(x, edge_index_in, edge_weight_in, edge_index_out, edge_weight_out,
           Wmi0, Wmo0, Ws0, bmi0, bmo0, bsi0, bso0, Cin0, Cout0,
           Wmi1, Wmo1, Ws1, bmi1, bmo1, bsi1, bso1, Cin1, Cout1,
           W_dec, b_dec):
    si2 = edge_index_in[0].reshape(_NCH, _C)
    di2 = edge_index_in[1].reshape(_NCH, _C)
    so2 = edge_index_out[0].reshape(_NCH, _C)
    do2 = edge_index_out[1].reshape(_NCH, _C)

    bmi0r, bmo0r = bmi0.reshape(1, _D), bmo0.reshape(1, _D)
    bsi0r, bso0r = bsi0.reshape(1, _D), bso0.reshape(1, _D)
    bmi1r, bmo1r = bmi1.reshape(1, _D), bmo1.reshape(1, _D)
    bsi1r, bso1r = bsi1.reshape(1, _D), bso1.reshape(1, _D)
    bdecr = b_dec.reshape(1, _CLS)

    hmi0a, hmi0b, hmo0a, hmo0b, sh0 = _tc_pre(x, Wmi0, Wmo0, Ws0)
    pin0a, pin0b, pout0a, pout0b = _sc_propagate(
        hmi0a, hmi0b, hmo0a, hmo0b, si2, di2, edge_weight_in,
        so2, do2, edge_weight_out)
    hmi1a, hmi1b, hmo1a, hmo1b, sh1 = _tc_mid(
        pin0a, pin0b, pout0a, pout0b, sh0, bmi0r, bmo0r, bsi0r, bso0r,
        Cin0, Cout0, Wmi1, Wmo1, Ws1)
    pin1a, pin1b, pout1a, pout1b = _sc_propagate(
        hmi1a, hmi1b, hmo1a, hmo1b, si2, di2, edge_weight_in,
        so2, do2, edge_weight_out)
    logp, emb = _tc_post(
        pin1a, pin1b, pout1a, pout1b, sh1, bmi1r, bmo1r, bsi1r, bso1r,
        Cin1, Cout1, W_dec, bdecr)
    return (logp, emb)
